# baseline (speedup 1.0000x reference)
import functools

import jax
import jax.numpy as jnp
from jax import lax
from jax.experimental import pallas as pl
from jax.experimental.pallas import tpu as pltpu
from jax.experimental.pallas import tpu_sc as plsc

N = 100000
E = 6400000
L = 3

LANES = 16
TILES = 16
NP = 102400
EPT = E // TILES
CH = 4000
CHUNKS = EPT // CH
NBUF = 4
SLICE = NP // TILES
VPT = SLICE // LANES


def _rsqrt(x):
  i = lax.bitcast_convert_type(x, jnp.int32)
  i = jnp.int32(0x5F3759DF) - lax.shift_right_logical(i, 1)
  y = lax.bitcast_convert_type(i, jnp.float32)
  for _ in range(3):
    y = y * (1.5 - 0.5 * x * y * y)
  return y


def _sc_body(h_hbm, src_hbm, dst_hbm, out_hbm,
             deg_o_sp, deg_i_sp, p_sp, agg_sp, c_sp,
             src_b, dst_b, val_b, ones_buf,
             agg_loc, na, nb, np_, cbuf, red_buf,
             idx_sems, sc_sems):
  cid = lax.axis_index("c")
  sid = lax.axis_index("s")

  @pl.when(cid == 0)
  def _core0():
    ebase = sid * EPT
    nbase = sid * SLICE
    zeros16 = jnp.zeros((LANES,), jnp.float32)
    ones16 = jnp.ones((LANES,), jnp.float32)

    def issue_idx(g, slot):
      off = ebase + g * CH
      pltpu.async_copy(src_hbm.at[pl.ds(off, CH)], src_b[slot],
                       idx_sems[slot])
      pltpu.async_copy(dst_hbm.at[pl.ds(off, CH)], dst_b[slot],
                       idx_sems[slot])

    def wait_idx(slot):
      pltpu.make_async_copy(src_hbm.at[pl.ds(0, CH)], src_b[slot],
                            idx_sems[slot]).wait()
      pltpu.make_async_copy(dst_hbm.at[pl.ds(0, CH)], dst_b[slot],
                            idx_sems[slot]).wait()

    def fill_ones(i, _):
      ones_buf[pl.ds(i * LANES, LANES)] = ones16
      return 0
    lax.fori_loop(0, CH // LANES, fill_ones, 0)

    def fill_zero(i, _):
      agg_loc[pl.ds(i * LANES, LANES)] = zeros16
      return 0
    lax.fori_loop(0, VPT, fill_zero, 0)

    pltpu.sync_copy(agg_loc, deg_o_sp.at[pl.ds(nbase, SLICE)])
    pltpu.sync_copy(agg_loc, deg_i_sp.at[pl.ds(nbase, SLICE)])
    pltpu.sync_copy(agg_loc, agg_sp.at[pl.ds(nbase, SLICE)])
    pltpu.sync_copy(agg_loc, p_sp.at[pl.ds(nbase, SLICE)])
    plsc.subcore_barrier()

    issue_idx(0, 0)

    def deg_iter(gi, _):
      for u in range(NBUF):
        g = gi * NBUF + u
        un = (u + 1) % NBUF

        @pl.when(g + 1 < CHUNKS)
        def _prefetch():
          issue_idx(g + 1, un)

        wait_idx(u)
        pltpu.sync_copy(ones_buf, deg_o_sp.at[src_b[u]], add=True)
        pltpu.sync_copy(ones_buf, deg_i_sp.at[dst_b[u]], add=True)
      return 0
    lax.fori_loop(0, CHUNKS // NBUF, deg_iter, 0)
    plsc.subcore_barrier()

    pltpu.sync_copy(h_hbm.at[pl.ds(nbase, SLICE)], na)
    pltpu.sync_copy(deg_o_sp.at[pl.ds(nbase, SLICE)], nb)

    def prep_o(i, _):
      s = pl.ds(i * LANES, LANES)
      no = _rsqrt(jnp.maximum(nb[s], 1.0))
      nb[s] = no
      np_[s] = na[s] * no
      return 0
    lax.fori_loop(0, VPT, prep_o, 0)
    pltpu.sync_copy(nb, deg_o_sp.at[pl.ds(nbase, SLICE)])
    pltpu.sync_copy(np_, p_sp.at[pl.ds(nbase, SLICE)])

    pltpu.sync_copy(deg_i_sp.at[pl.ds(nbase, SLICE)], nb)

    def prep_i(i, _):
      s = pl.ds(i * LANES, LANES)
      nb[s] = _rsqrt(jnp.maximum(nb[s], 1.0))
      return 0
    lax.fori_loop(0, VPT, prep_i, 0)
    pltpu.sync_copy(nb, deg_i_sp.at[pl.ds(nbase, SLICE)])
    plsc.subcore_barrier()

    for layer in range(L):
      issue_idx(0, 0)

      def edge_iter(gi, _):
        for u in range(NBUF):
          g = gi * NBUF + u
          un = (u + 1) % NBUF

          @pl.when(g + 1 < CHUNKS)
          def _prefetch():
            issue_idx(g + 1, un)

          wait_idx(u)
          pltpu.sync_copy(p_sp.at[src_b[u]], val_b[u])
          pltpu.sync_copy(val_b[u], agg_sp.at[dst_b[u]], add=True)
        return 0
      lax.fori_loop(0, CHUNKS // NBUF, edge_iter, 0)
      plsc.subcore_barrier()

      pltpu.sync_copy(agg_sp.at[pl.ds(nbase, SLICE)], agg_loc)
      pltpu.sync_copy(deg_i_sp.at[pl.ds(nbase, SLICE)], na)
      pltpu.sync_copy(deg_o_sp.at[pl.ds(nbase, SLICE)], nb)

      def node(i, c):
        s = pl.ds(i * LANES, LANES)
        hn = agg_loc[s] * na[s]
        np_[s] = hn * nb[s]
        agg_loc[s] = zeros16
        return c + hn * hn
      c = lax.fori_loop(0, VPT, node, zeros16)
      cbuf[...] = c
      pltpu.sync_copy(cbuf, c_sp.at[pl.ds(sid * LANES, LANES)])
      pltpu.sync_copy(np_, p_sp.at[pl.ds(nbase, SLICE)])
      pltpu.sync_copy(agg_loc, agg_sp.at[pl.ds(nbase, SLICE)])
      plsc.subcore_barrier()

      @pl.when(sid == 0)
      def _reduce():
        pltpu.sync_copy(c_sp, red_buf)
        acc = zeros16
        for r in range(TILES):
          acc = acc + red_buf[pl.ds(r * LANES, LANES)]
        cbuf[...] = acc
        pltpu.sync_copy(cbuf, out_hbm.at[pl.ds(layer * LANES, LANES)])


@functools.partial(jax.jit, static_argnums=())
def _sc_call(h1, src1d, dst1d):
  mesh = plsc.VectorSubcoreMesh(core_axis_name="c", subcore_axis_name="s")
  f = pl.kernel(
      _sc_body,
      out_type=jax.ShapeDtypeStruct((L * LANES,), jnp.float32),
      mesh=mesh,
      scratch_types=[
          pltpu.VMEM_SHARED((NP,), jnp.float32),
          pltpu.VMEM_SHARED((NP,), jnp.float32),
          pltpu.VMEM_SHARED((NP,), jnp.float32),
          pltpu.VMEM_SHARED((NP,), jnp.float32),
          pltpu.VMEM_SHARED((TILES * LANES,), jnp.float32),
          [pltpu.VMEM((CH,), jnp.int32)] * NBUF,
          [pltpu.VMEM((CH,), jnp.int32)] * NBUF,
          [pltpu.VMEM((CH,), jnp.float32)] * NBUF,
          pltpu.VMEM((CH,), jnp.float32),
          pltpu.VMEM((SLICE,), jnp.float32),
          pltpu.VMEM((SLICE,), jnp.float32),
          pltpu.VMEM((SLICE,), jnp.float32),
          pltpu.VMEM((SLICE,), jnp.float32),
          pltpu.VMEM((LANES,), jnp.float32),
          pltpu.VMEM((TILES * LANES,), jnp.float32),
          [pltpu.SemaphoreType.DMA] * NBUF,
          [pltpu.SemaphoreType.DMA] * NBUF,
      ],
  )
  return f(h1, src1d, dst1d)


def kernel(h, edge_index):
  h1 = jnp.pad(h[:, 0], (0, NP - N))
  out = _sc_call(h1, edge_index[0], edge_index[1])
  return jnp.sum(out.reshape(L, LANES), axis=1)


# speedup vs baseline: 1.0471x; 1.0471x over previous
import functools

import jax
import jax.numpy as jnp
from jax import lax
from jax.experimental import pallas as pl
from jax.experimental.pallas import tpu as pltpu
from jax.experimental.pallas import tpu_sc as plsc

N = 100000
E = 6400000
L = 3

LANES = 16
TILES = 16
NP = 102400
EPT = E // TILES
CH = 2000
CHUNKS = EPT // CH
NBUF = 4
SLICE = NP // TILES
VPT = SLICE // LANES


def _rsqrt(x):
  i = lax.bitcast_convert_type(x, jnp.int32)
  i = jnp.int32(0x5F3759DF) - lax.shift_right_logical(i, 1)
  y = lax.bitcast_convert_type(i, jnp.float32)
  for _ in range(3):
    y = y * (1.5 - 0.5 * x * y * y)
  return y


def _sc_body(h_hbm, src_hbm, dst_hbm, out_hbm,
             deg_o_sp, deg_i_sp, p_sp, agg_sp, c_sp,
             src_b, dst_b, val_b, ones_buf,
             agg_loc, na, nb, np_, cbuf, red_buf,
             idx_sems, sc_sems, g_sems):
  cid = lax.axis_index("c")
  sid = lax.axis_index("s")

  @pl.when(cid == 0)
  def _core0():
    ebase = sid * EPT
    nbase = sid * SLICE
    zeros16 = jnp.zeros((LANES,), jnp.float32)
    ones16 = jnp.ones((LANES,), jnp.float32)

    def issue_idx(g, slot):
      off = ebase + g * CH
      pltpu.async_copy(src_hbm.at[pl.ds(off, CH)], src_b[slot],
                       idx_sems[slot])
      pltpu.async_copy(dst_hbm.at[pl.ds(off, CH)], dst_b[slot],
                       idx_sems[slot])

    def wait_idx(slot):
      pltpu.make_async_copy(src_hbm.at[pl.ds(0, CH)], src_b[slot],
                            idx_sems[slot]).wait()
      pltpu.make_async_copy(dst_hbm.at[pl.ds(0, CH)], dst_b[slot],
                            idx_sems[slot]).wait()

    def fill_ones(i, _):
      ones_buf[pl.ds(i * LANES, LANES)] = ones16
      return 0
    lax.fori_loop(0, CH // LANES, fill_ones, 0)

    def fill_zero(i, _):
      agg_loc[pl.ds(i * LANES, LANES)] = zeros16
      return 0
    lax.fori_loop(0, VPT, fill_zero, 0)

    pltpu.sync_copy(agg_loc, deg_o_sp.at[pl.ds(nbase, SLICE)])
    pltpu.sync_copy(agg_loc, deg_i_sp.at[pl.ds(nbase, SLICE)])
    pltpu.sync_copy(agg_loc, agg_sp.at[pl.ds(nbase, SLICE)])
    pltpu.sync_copy(agg_loc, p_sp.at[pl.ds(nbase, SLICE)])
    plsc.subcore_barrier()

    issue_idx(0, 0)

    def deg_iter(gi, _):
      for u in range(NBUF):
        g = gi * NBUF + u
        un = (u + 1) % NBUF

        @pl.when(g + 1 < CHUNKS)
        def _prefetch():
          issue_idx(g + 1, un)

        wait_idx(u)
        pltpu.sync_copy(ones_buf, deg_o_sp.at[src_b[u]], add=True)
        pltpu.sync_copy(ones_buf, deg_i_sp.at[dst_b[u]], add=True)
      return 0
    lax.fori_loop(0, CHUNKS // NBUF, deg_iter, 0)
    plsc.subcore_barrier()

    pltpu.sync_copy(h_hbm.at[pl.ds(nbase, SLICE)], na)
    pltpu.sync_copy(deg_o_sp.at[pl.ds(nbase, SLICE)], nb)

    def prep_o(i, _):
      s = pl.ds(i * LANES, LANES)
      no = _rsqrt(jnp.maximum(nb[s], 1.0))
      nb[s] = no
      np_[s] = na[s] * no
      return 0
    lax.fori_loop(0, VPT, prep_o, 0)
    pltpu.sync_copy(nb, deg_o_sp.at[pl.ds(nbase, SLICE)])
    pltpu.sync_copy(np_, p_sp.at[pl.ds(nbase, SLICE)])

    pltpu.sync_copy(deg_i_sp.at[pl.ds(nbase, SLICE)], nb)

    def prep_i(i, _):
      s = pl.ds(i * LANES, LANES)
      nb[s] = _rsqrt(jnp.maximum(nb[s], 1.0))
      return 0
    lax.fori_loop(0, VPT, prep_i, 0)
    pltpu.sync_copy(nb, deg_i_sp.at[pl.ds(nbase, SLICE)])
    plsc.subcore_barrier()

    NB = CHUNKS // NBUF

    def issue_batch_idx(b, r):
      for u in range(NBUF):
        off = ebase + (b * NBUF + u) * CH
        pltpu.async_copy(src_hbm.at[pl.ds(off, CH)], src_b[u], idx_sems[u])
        pltpu.async_copy(dst_hbm.at[pl.ds(off, CH)], dst_b[r * NBUF + u],
                         idx_sems[u])

    def wait_batch_idx(r):
      for u in range(NBUF):
        pltpu.make_async_copy(src_hbm.at[pl.ds(0, CH)], src_b[u],
                              idx_sems[u]).wait()
        pltpu.make_async_copy(dst_hbm.at[pl.ds(0, CH)],
                              dst_b[r * NBUF + u], idx_sems[u]).wait()

    def wait_batch_scatter(r):
      for u in range(NBUF):
        k = r * NBUF + u
        pltpu.make_async_copy(val_b[k], agg_sp.at[dst_b[k]],
                              sc_sems[k]).wait()

    for layer in range(L):
      issue_batch_idx(0, 0)

      def edge_batch(b2, _):
        for r in range(2):
          b = b2 * 2 + r
          wait_batch_idx(r)
          for u in range(NBUF):
            k = r * NBUF + u
            pltpu.async_copy(p_sp.at[src_b[u]], val_b[k], g_sems[u])
          for u in range(NBUF):
            k = r * NBUF + u
            pltpu.make_async_copy(p_sp.at[src_b[u]], val_b[k],
                                  g_sems[u]).wait()
          for u in range(NBUF):
            k = r * NBUF + u
            pltpu.async_copy(val_b[k], agg_sp.at[dst_b[k]], sc_sems[k],
                             add=True)
          @pl.when(b >= 1)
          def _wait_prev():
            wait_batch_scatter(1 - r)
          @pl.when(b + 1 < NB)
          def _prefetch():
            issue_batch_idx(b + 1, 1 - r)
        return 0

      lax.fori_loop(0, NB // 2, edge_batch, 0)
      wait_batch_scatter((NB - 1) % 2)
      plsc.subcore_barrier()

      pltpu.sync_copy(agg_sp.at[pl.ds(nbase, SLICE)], agg_loc)
      pltpu.sync_copy(deg_i_sp.at[pl.ds(nbase, SLICE)], na)
      pltpu.sync_copy(deg_o_sp.at[pl.ds(nbase, SLICE)], nb)

      def node(i, c):
        s = pl.ds(i * LANES, LANES)
        hn = agg_loc[s] * na[s]
        np_[s] = hn * nb[s]
        agg_loc[s] = zeros16
        return c + hn * hn
      c = lax.fori_loop(0, VPT, node, zeros16)
      cbuf[...] = c
      pltpu.sync_copy(cbuf, c_sp.at[pl.ds(sid * LANES, LANES)])
      pltpu.sync_copy(np_, p_sp.at[pl.ds(nbase, SLICE)])
      pltpu.sync_copy(agg_loc, agg_sp.at[pl.ds(nbase, SLICE)])
      plsc.subcore_barrier()

      @pl.when(sid == 0)
      def _reduce():
        pltpu.sync_copy(c_sp, red_buf)
        acc = zeros16
        for r in range(TILES):
          acc = acc + red_buf[pl.ds(r * LANES, LANES)]
        cbuf[...] = acc
        pltpu.sync_copy(cbuf, out_hbm.at[pl.ds(layer * LANES, LANES)])


@functools.partial(jax.jit, static_argnums=())
def _sc_call(h1, src1d, dst1d):
  mesh = plsc.VectorSubcoreMesh(core_axis_name="c", subcore_axis_name="s")
  f = pl.kernel(
      _sc_body,
      out_type=jax.ShapeDtypeStruct((L * LANES,), jnp.float32),
      mesh=mesh,
      scratch_types=[
          pltpu.VMEM_SHARED((NP,), jnp.float32),
          pltpu.VMEM_SHARED((NP,), jnp.float32),
          pltpu.VMEM_SHARED((NP,), jnp.float32),
          pltpu.VMEM_SHARED((NP,), jnp.float32),
          pltpu.VMEM_SHARED((TILES * LANES,), jnp.float32),
          [pltpu.VMEM((CH,), jnp.int32)] * NBUF,
          [pltpu.VMEM((CH,), jnp.int32)] * (2 * NBUF),
          [pltpu.VMEM((CH,), jnp.float32)] * (2 * NBUF),
          pltpu.VMEM((CH,), jnp.float32),
          pltpu.VMEM((SLICE,), jnp.float32),
          pltpu.VMEM((SLICE,), jnp.float32),
          pltpu.VMEM((SLICE,), jnp.float32),
          pltpu.VMEM((SLICE,), jnp.float32),
          pltpu.VMEM((LANES,), jnp.float32),
          pltpu.VMEM((TILES * LANES,), jnp.float32),
          [pltpu.SemaphoreType.DMA] * NBUF,
          [pltpu.SemaphoreType.DMA] * (2 * NBUF),
          [pltpu.SemaphoreType.DMA] * NBUF,
      ],
  )
  return f(h1, src1d, dst1d)


def kernel(h, edge_index):
  h1 = jnp.pad(h[:, 0], (0, NP - N))
  out = _sc_call(h1, edge_index[0], edge_index[1])
  return jnp.sum(out.reshape(L, LANES), axis=1)


# speedup vs baseline: 1.7156x; 1.6384x over previous
import functools

import jax
import jax.numpy as jnp
from jax import lax
from jax.experimental import pallas as pl
from jax.experimental.pallas import tpu as pltpu
from jax.experimental.pallas import tpu_sc as plsc

N = 100000
E = 6400000
L = 3

LANES = 16
TILES = 16
NP = 102400
EPT = E // TILES
CH = 2000
CHUNKS = EPT // CH
NBUF = 4
SLICE = NP // TILES
SUB = 800
NSUB = SLICE // SUB
VPS = SUB // LANES
GU = 5


def _rsqrt(x):
  i = lax.bitcast_convert_type(x, jnp.int32)
  i = jnp.int32(0x5F3759DF) - lax.shift_right_logical(i, 1)
  y = lax.bitcast_convert_type(i, jnp.float32)
  for _ in range(3):
    y = y * (1.5 - 0.5 * x * y * y)
  return y


def _sc_body(h_hbm, src_hbm, dst_hbm, c_out, no_out, ni_out,
             tab_sp, c_sp,
             src_b, dst_b, val_b, p_repl, cbuf,
             idx_sems, sc_sems, v_sems):
  cid = lax.axis_index("c")
  sid = lax.axis_index("s")

  @pl.when(cid == 0)
  def _core0():
    ebase = sid * EPT
    nbase = sid * SLICE
    zeros16 = jnp.zeros((LANES,), jnp.float32)
    ones16 = jnp.ones((LANES,), jnp.float32)

    def fill(ref, words, vec):
      def body(i, _):
        ref[pl.ds(i * LANES, LANES)] = vec
        return 0
      lax.fori_loop(0, words // LANES, body, 0)

    def zero_my_slice(zref):
      for q in range(NSUB):
        pltpu.sync_copy(zref, tab_sp.at[pl.ds(nbase + q * SUB, SUB)])

    def ones_pass(ix_hbm):
      pltpu.async_copy(ix_hbm.at[pl.ds(ebase, CH)], dst_b[0], idx_sems[0])

      def deg_iter(gi, _):
        for u in range(NBUF):
          g = gi * NBUF + u
          un = (u + 1) % NBUF

          @pl.when(g >= NBUF - 1)
          def _reclaim():
            pltpu.make_async_copy(val_b[0], tab_sp.at[dst_b[un]],
                                  sc_sems[un]).wait()

          @pl.when(g + 1 < CHUNKS)
          def _prefetch():
            pltpu.async_copy(ix_hbm.at[pl.ds(ebase + (g + 1) * CH, CH)],
                             dst_b[un], idx_sems[un])

          pltpu.make_async_copy(ix_hbm.at[pl.ds(0, CH)], dst_b[u],
                                idx_sems[u]).wait()
          pltpu.async_copy(val_b[0], tab_sp.at[dst_b[u]], sc_sems[u],
                           add=True)
        return 0
      lax.fori_loop(0, CHUNKS // NBUF, deg_iter, 0)
      for s in range(1, NBUF):
        pltpu.make_async_copy(val_b[0], tab_sp.at[dst_b[s]],
                              sc_sems[s]).wait()

    fill(val_b[0], CH, ones16)
    fill(val_b[1], 2 * SUB, zeros16)
    zero_my_slice(val_b[1].at[pl.ds(SUB, SUB)])
    plsc.subcore_barrier()

    ones_pass(dst_hbm)
    plsc.subcore_barrier()
    for q in range(NSUB):
      off = nbase + q * SUB
      pltpu.sync_copy(tab_sp.at[pl.ds(off, SUB)], val_b[1].at[pl.ds(0, SUB)])

      def prep_i(i, _):
        s = pl.ds(i * LANES, LANES)
        val_b[1][s] = _rsqrt(jnp.maximum(val_b[1][s], 1.0))
        return 0
      lax.fori_loop(0, VPS, prep_i, 0)
      pltpu.sync_copy(val_b[1].at[pl.ds(0, SUB)], ni_out.at[pl.ds(off, SUB)])
      pltpu.sync_copy(val_b[1].at[pl.ds(SUB, SUB)],
                      tab_sp.at[pl.ds(off, SUB)])
    plsc.subcore_barrier()

    ones_pass(src_hbm)
    plsc.subcore_barrier()
    for q in range(NSUB):
      off = nbase + q * SUB
      pltpu.sync_copy(tab_sp.at[pl.ds(off, SUB)], val_b[1].at[pl.ds(0, SUB)])
      pltpu.sync_copy(h_hbm.at[pl.ds(off, SUB)], val_b[0].at[pl.ds(0, SUB)])

      def prep_o(i, _):
        s = pl.ds(i * LANES, LANES)
        ss = pl.ds(SUB + (i * LANES), LANES)
        no = _rsqrt(jnp.maximum(val_b[1][s], 1.0))
        val_b[1][s] = no
        val_b[0][ss] = val_b[0][s] * no
        return 0
      lax.fori_loop(0, VPS, prep_o, 0)
      pltpu.sync_copy(val_b[1].at[pl.ds(0, SUB)], no_out.at[pl.ds(off, SUB)])
      pltpu.sync_copy(val_b[0].at[pl.ds(SUB, SUB)],
                      tab_sp.at[pl.ds(off, SUB)])
    plsc.subcore_barrier()

    for layer in range(L):
      pltpu.sync_copy(tab_sp.at[pl.ds(0, N)], p_repl)
      plsc.subcore_barrier()
      fill(val_b[1], SUB, zeros16)
      zero_my_slice(val_b[1].at[pl.ds(0, SUB)])
      plsc.subcore_barrier()

      pltpu.async_copy(src_hbm.at[pl.ds(ebase, CH)], src_b[0], idx_sems[0])
      pltpu.async_copy(dst_hbm.at[pl.ds(ebase, CH)], dst_b[0], idx_sems[0])

      def edge_iter(gi, _):
        for u in range(NBUF):
          g = gi * NBUF + u
          un = (u + 1) % NBUF
          v = u % 2

          @pl.when(g >= 2)
          def _reclaim():
            pltpu.make_async_copy(val_b[v], tab_sp.at[dst_b[(u + 2) % 4]],
                                  v_sems[v]).wait()

          @pl.when(g + 1 < CHUNKS)
          def _prefetch():
            off = ebase + (g + 1) * CH
            pltpu.async_copy(src_hbm.at[pl.ds(off, CH)], src_b[un],
                             idx_sems[un])
            pltpu.async_copy(dst_hbm.at[pl.ds(off, CH)], dst_b[un],
                             idx_sems[un])

          pltpu.make_async_copy(src_hbm.at[pl.ds(0, CH)], src_b[u],
                                idx_sems[u]).wait()
          pltpu.make_async_copy(dst_hbm.at[pl.ds(0, CH)], dst_b[u],
                                idx_sems[u]).wait()

          def vgather(j, _):
            for t in range(GU):
              s = pl.ds((j * GU + t) * LANES, LANES)
              val_b[v][s] = plsc.load_gather(p_repl, [src_b[u][s]])
            return 0
          lax.fori_loop(0, CH // (LANES * GU), vgather, 0)

          pltpu.async_copy(val_b[v], tab_sp.at[dst_b[u]], v_sems[v],
                           add=True)
        return 0
      lax.fori_loop(0, CHUNKS // NBUF, edge_iter, 0)
      for v in range(2):
        pltpu.make_async_copy(val_b[v], tab_sp.at[dst_b[2 + v]],
                              v_sems[v]).wait()
      plsc.subcore_barrier()

      c = zeros16
      for q in range(NSUB):
        off = nbase + q * SUB
        pltpu.sync_copy(tab_sp.at[pl.ds(off, SUB)], val_b[0].at[pl.ds(0, SUB)])
        pltpu.sync_copy(ni_out.at[pl.ds(off, SUB)],
                        val_b[0].at[pl.ds(SUB, SUB)])
        pltpu.sync_copy(no_out.at[pl.ds(off, SUB)],
                        val_b[1].at[pl.ds(0, SUB)])

        def node(i, cc):
          s = pl.ds(i * LANES, LANES)
          ss = pl.ds(SUB + (i * LANES), LANES)
          hn = val_b[0][s] * val_b[0][ss]
          val_b[1][ss] = hn * val_b[1][s]
          return cc + hn * hn
        c = lax.fori_loop(0, VPS, node, c)
        pltpu.sync_copy(val_b[1].at[pl.ds(SUB, SUB)],
                        tab_sp.at[pl.ds(off, SUB)])
      cbuf[...] = c
      pltpu.sync_copy(cbuf, c_sp.at[pl.ds(sid * LANES, LANES)])
      plsc.subcore_barrier()

      @pl.when(sid == 0)
      def _reduce():
        pltpu.sync_copy(c_sp, val_b[0].at[pl.ds(0, TILES * LANES)])
        acc = zeros16
        for r in range(TILES):
          acc = acc + val_b[0][pl.ds(r * LANES, LANES)]
        cbuf[...] = acc
        pltpu.sync_copy(cbuf, c_out.at[pl.ds(layer * LANES, LANES)])


@functools.partial(jax.jit, static_argnums=())
def _sc_call(h1, src1d, dst1d):
  mesh = plsc.VectorSubcoreMesh(core_axis_name="c", subcore_axis_name="s")
  f = pl.kernel(
      _sc_body,
      out_type=(
          jax.ShapeDtypeStruct((L * LANES,), jnp.float32),
          jax.ShapeDtypeStruct((NP,), jnp.float32),
          jax.ShapeDtypeStruct((NP,), jnp.float32),
      ),
      mesh=mesh,
      compiler_params=pltpu.CompilerParams(needs_layout_passes=False),
      scratch_types=[
          pltpu.VMEM_SHARED((NP,), jnp.float32),
          pltpu.VMEM_SHARED((TILES * LANES,), jnp.float32),
          [pltpu.VMEM((CH,), jnp.int32)] * NBUF,
          [pltpu.VMEM((CH,), jnp.int32)] * NBUF,
          [pltpu.VMEM((CH,), jnp.float32)] * 2,
          pltpu.VMEM((N,), jnp.float32),
          pltpu.VMEM((LANES,), jnp.float32),
          [pltpu.SemaphoreType.DMA] * NBUF,
          [pltpu.SemaphoreType.DMA] * NBUF,
          [pltpu.SemaphoreType.DMA] * 2,
      ],
  )
  return f(h1, src1d, dst1d)


def kernel(h, edge_index):
  h1 = jnp.pad(h[:, 0], (0, NP - N))
  out, _, _ = _sc_call(h1, edge_index[0], edge_index[1])
  return jnp.sum(out.reshape(L, LANES), axis=1)


# speedup vs baseline: 1.7178x; 1.0013x over previous
import functools

import jax
import jax.numpy as jnp
from jax import lax
from jax.experimental import pallas as pl
from jax.experimental.pallas import tpu as pltpu
from jax.experimental.pallas import tpu_sc as plsc

N = 100000
E = 6400000
L = 3

LANES = 16
TILES = 16
NP = 102400
EPT = E // TILES
CH = 2000
CHUNKS = EPT // CH
NBUF = 4
SLICE = NP // TILES
SUB = 800
NSUB = SLICE // SUB
VPS = SUB // LANES
GU = 5


def _node_scope():
  with jax.named_scope("node"):
    yield from range(NSUB)


def _rsqrt(x):
  i = lax.bitcast_convert_type(x, jnp.int32)
  i = jnp.int32(0x5F3759DF) - lax.shift_right_logical(i, 1)
  y = lax.bitcast_convert_type(i, jnp.float32)
  for _ in range(3):
    y = y * (1.5 - 0.5 * x * y * y)
  return y


def _sc_body(h_hbm, src_hbm, dst_hbm, c_out, no_out, ni_out,
             tab_sp, c_sp,
             src_b, dst_b, val_b, p_repl, cbuf,
             idx_sems, sc_sems, v_sems):
  cid = lax.axis_index("c")
  sid = lax.axis_index("s")

  @pl.when(cid == 0)
  def _core0():
    ebase = sid * EPT
    nbase = sid * SLICE
    zeros16 = jnp.zeros((LANES,), jnp.float32)
    ones16 = jnp.ones((LANES,), jnp.float32)

    def fill(ref, words, vec):
      def body(i, _):
        ref[pl.ds(i * LANES, LANES)] = vec
        return 0
      lax.fori_loop(0, words // LANES, body, 0)

    def zero_my_slice(zref):
      for q in range(NSUB):
        pltpu.sync_copy(zref, tab_sp.at[pl.ds(nbase + q * SUB, SUB)])

    def ones_pass(ix_hbm):
      pltpu.async_copy(ix_hbm.at[pl.ds(ebase, CH)], dst_b[0], idx_sems[0])

      def deg_iter(gi, _):
        for u in range(NBUF):
          g = gi * NBUF + u
          un = (u + 1) % NBUF

          @pl.when(g >= NBUF - 1)
          def _reclaim():
            pltpu.make_async_copy(val_b[0], tab_sp.at[dst_b[un]],
                                  sc_sems[un]).wait()

          @pl.when(g + 1 < CHUNKS)
          def _prefetch():
            pltpu.async_copy(ix_hbm.at[pl.ds(ebase + (g + 1) * CH, CH)],
                             dst_b[un], idx_sems[un])

          pltpu.make_async_copy(ix_hbm.at[pl.ds(0, CH)], dst_b[u],
                                idx_sems[u]).wait()
          pltpu.async_copy(val_b[0], tab_sp.at[dst_b[u]], sc_sems[u],
                           add=True)
        return 0
      lax.fori_loop(0, CHUNKS // NBUF, deg_iter, 0)
      for s in range(1, NBUF):
        pltpu.make_async_copy(val_b[0], tab_sp.at[dst_b[s]],
                              sc_sems[s]).wait()

    fill(val_b[0], CH, ones16)
    fill(val_b[1], 2 * SUB, zeros16)
    zero_my_slice(val_b[1].at[pl.ds(SUB, SUB)])
    plsc.subcore_barrier()

    with jax.named_scope("deg_in"):
      ones_pass(dst_hbm)
    plsc.subcore_barrier()
    for q in range(NSUB):
      off = nbase + q * SUB
      pltpu.sync_copy(tab_sp.at[pl.ds(off, SUB)], val_b[1].at[pl.ds(0, SUB)])

      def prep_i(i, _):
        s = pl.ds(i * LANES, LANES)
        val_b[1][s] = _rsqrt(jnp.maximum(val_b[1][s], 1.0))
        return 0
      lax.fori_loop(0, VPS, prep_i, 0)
      pltpu.sync_copy(val_b[1].at[pl.ds(0, SUB)], ni_out.at[pl.ds(off, SUB)])
      pltpu.sync_copy(val_b[1].at[pl.ds(SUB, SUB)],
                      tab_sp.at[pl.ds(off, SUB)])
    plsc.subcore_barrier()

    with jax.named_scope("deg_out"):
      ones_pass(src_hbm)
    plsc.subcore_barrier()
    for q in range(NSUB):
      off = nbase + q * SUB
      pltpu.sync_copy(tab_sp.at[pl.ds(off, SUB)], val_b[1].at[pl.ds(0, SUB)])
      pltpu.sync_copy(h_hbm.at[pl.ds(off, SUB)], val_b[0].at[pl.ds(0, SUB)])

      def prep_o(i, _):
        s = pl.ds(i * LANES, LANES)
        ss = pl.ds(SUB + (i * LANES), LANES)
        no = _rsqrt(jnp.maximum(val_b[1][s], 1.0))
        val_b[1][s] = no
        val_b[0][ss] = val_b[0][s] * no
        return 0
      lax.fori_loop(0, VPS, prep_o, 0)
      pltpu.sync_copy(val_b[1].at[pl.ds(0, SUB)], no_out.at[pl.ds(off, SUB)])
      pltpu.sync_copy(val_b[0].at[pl.ds(SUB, SUB)],
                      tab_sp.at[pl.ds(off, SUB)])
    plsc.subcore_barrier()

    for layer in range(L):
      with jax.named_scope("snapshot"):
        pltpu.sync_copy(tab_sp.at[pl.ds(0, N)], p_repl)
      plsc.subcore_barrier()
      with jax.named_scope("zero"):
        fill(val_b[1], SUB, zeros16)
        zero_my_slice(val_b[1].at[pl.ds(0, SUB)])
      plsc.subcore_barrier()

      pltpu.async_copy(src_hbm.at[pl.ds(ebase, CH)], src_b[0], idx_sems[0])
      pltpu.async_copy(dst_hbm.at[pl.ds(ebase, CH)], dst_b[0], idx_sems[0])

      def edge_iter(gi, _):
        for u in range(NBUF):
          g = gi * NBUF + u
          un = (u + 1) % NBUF
          v = u % 2

          @pl.when(g >= 2)
          def _reclaim():
            pltpu.make_async_copy(val_b[v], tab_sp.at[dst_b[(u + 2) % 4]],
                                  v_sems[v]).wait()

          @pl.when(g + 1 < CHUNKS)
          def _prefetch():
            off = ebase + (g + 1) * CH
            pltpu.async_copy(src_hbm.at[pl.ds(off, CH)], src_b[un],
                             idx_sems[un])
            pltpu.async_copy(dst_hbm.at[pl.ds(off, CH)], dst_b[un],
                             idx_sems[un])

          pltpu.make_async_copy(src_hbm.at[pl.ds(0, CH)], src_b[u],
                                idx_sems[u]).wait()
          pltpu.make_async_copy(dst_hbm.at[pl.ds(0, CH)], dst_b[u],
                                idx_sems[u]).wait()

          def vgather(j, _):
            for t in range(GU):
              s = pl.ds((j * GU + t) * LANES, LANES)
              val_b[v][s] = plsc.load_gather(p_repl, [src_b[u][s]])
            return 0
          lax.fori_loop(0, CH // (LANES * GU), vgather, 0)

          pltpu.async_copy(val_b[v], tab_sp.at[dst_b[u]], v_sems[v],
                           add=True)
        return 0
      with jax.named_scope("edges"):
        lax.fori_loop(0, CHUNKS // NBUF, edge_iter, 0)
        for v in range(2):
          pltpu.make_async_copy(val_b[v], tab_sp.at[dst_b[2 + v]],
                                v_sems[v]).wait()
      plsc.subcore_barrier()

      c = zeros16
      for q in _node_scope():
        off = nbase + q * SUB
        pltpu.sync_copy(tab_sp.at[pl.ds(off, SUB)], val_b[0].at[pl.ds(0, SUB)])
        pltpu.sync_copy(ni_out.at[pl.ds(off, SUB)],
                        val_b[0].at[pl.ds(SUB, SUB)])
        pltpu.sync_copy(no_out.at[pl.ds(off, SUB)],
                        val_b[1].at[pl.ds(0, SUB)])

        def node(i, cc):
          s = pl.ds(i * LANES, LANES)
          ss = pl.ds(SUB + (i * LANES), LANES)
          hn = val_b[0][s] * val_b[0][ss]
          val_b[1][ss] = hn * val_b[1][s]
          return cc + hn * hn
        c = lax.fori_loop(0, VPS, node, c)
        pltpu.sync_copy(val_b[1].at[pl.ds(SUB, SUB)],
                        tab_sp.at[pl.ds(off, SUB)])
      cbuf[...] = c
      pltpu.sync_copy(cbuf, c_sp.at[pl.ds(sid * LANES, LANES)])
      plsc.subcore_barrier()

      @pl.when(sid == 0)
      def _reduce():
        pltpu.sync_copy(c_sp, val_b[0].at[pl.ds(0, TILES * LANES)])
        acc = zeros16
        for r in range(TILES):
          acc = acc + val_b[0][pl.ds(r * LANES, LANES)]
        cbuf[...] = acc
        pltpu.sync_copy(cbuf, c_out.at[pl.ds(layer * LANES, LANES)])


@functools.partial(jax.jit, static_argnums=())
def _sc_call(h1, src1d, dst1d):
  mesh = plsc.VectorSubcoreMesh(core_axis_name="c", subcore_axis_name="s")
  f = pl.kernel(
      _sc_body,
      out_type=(
          jax.ShapeDtypeStruct((L * LANES,), jnp.float32),
          jax.ShapeDtypeStruct((NP,), jnp.float32),
          jax.ShapeDtypeStruct((NP,), jnp.float32),
      ),
      mesh=mesh,
      compiler_params=pltpu.CompilerParams(needs_layout_passes=False),
      scratch_types=[
          pltpu.VMEM_SHARED((NP,), jnp.float32),
          pltpu.VMEM_SHARED((TILES * LANES,), jnp.float32),
          [pltpu.VMEM((CH,), jnp.int32)] * NBUF,
          [pltpu.VMEM((CH,), jnp.int32)] * NBUF,
          [pltpu.VMEM((CH,), jnp.float32)] * 2,
          pltpu.VMEM((N,), jnp.float32),
          pltpu.VMEM((LANES,), jnp.float32),
          [pltpu.SemaphoreType.DMA] * NBUF,
          [pltpu.SemaphoreType.DMA] * NBUF,
          [pltpu.SemaphoreType.DMA] * 2,
      ],
  )
  return f(h1, src1d, dst1d)


def kernel(h, edge_index):
  h1 = jnp.pad(h[:, 0], (0, NP - N))
  out, _, _ = _sc_call(h1, edge_index[0], edge_index[1])
  return jnp.sum(out.reshape(L, LANES), axis=1)


# speedup vs baseline: 1.7251x; 1.0043x over previous
import functools

import jax
import jax.numpy as jnp
from jax import lax
from jax.experimental import pallas as pl
from jax.experimental.pallas import tpu as pltpu
from jax.experimental.pallas import tpu_sc as plsc

N = 100000
E = 6400000
L = 3

LANES = 16
TILES = 16
NP = 102400
EPT = E // TILES
CH = 2000
CHUNKS = EPT // CH
NBUF = 4
SLICE = NP // TILES
SUB = 800
NSUB = SLICE // SUB
VPS = SUB // LANES
GU = 5


def _rsqrt(x):
  i = lax.bitcast_convert_type(x, jnp.int32)
  i = jnp.int32(0x5F3759DF) - lax.shift_right_logical(i, 1)
  y = lax.bitcast_convert_type(i, jnp.float32)
  for _ in range(3):
    y = y * (1.5 - 0.5 * x * y * y)
  return y


def _sc_body(h_hbm, src_hbm, dst_hbm, c_out, no_out, ni_out,
             tab_sp, c_sp,
             src_b, dst_b, val_b, p_repl, cbuf,
             idx_sems, sc_sems, v_sems):
  cid = lax.axis_index("c")
  sid = lax.axis_index("s")

  @pl.when(cid == 0)
  def _core0():
    ebase = sid * EPT
    nbase = sid * SLICE
    zeros16 = jnp.zeros((LANES,), jnp.float32)
    ones16 = jnp.ones((LANES,), jnp.float32)

    def fill(ref, words, vec):
      def body(i, _):
        ref[pl.ds(i * LANES, LANES)] = vec
        return 0
      lax.fori_loop(0, words // LANES, body, 0)

    def zero_my_slice(zref):
      for q in range(NSUB):
        pltpu.sync_copy(zref, tab_sp.at[pl.ds(nbase + q * SUB, SUB)])

    def ones_pass(ix_hbm):
      pltpu.async_copy(ix_hbm.at[pl.ds(ebase, CH)], dst_b[0], idx_sems[0])

      def deg_iter(gi, _):
        for u in range(NBUF):
          g = gi * NBUF + u
          un = (u + 1) % NBUF

          @pl.when(g >= NBUF - 1)
          def _reclaim():
            pltpu.make_async_copy(val_b[0], tab_sp.at[dst_b[un]],
                                  sc_sems[un]).wait()

          @pl.when(g + 1 < CHUNKS)
          def _prefetch():
            pltpu.async_copy(ix_hbm.at[pl.ds(ebase + (g + 1) * CH, CH)],
                             dst_b[un], idx_sems[un])

          pltpu.make_async_copy(ix_hbm.at[pl.ds(0, CH)], dst_b[u],
                                idx_sems[u]).wait()
          pltpu.async_copy(val_b[0], tab_sp.at[dst_b[u]], sc_sems[u],
                           add=True)
        return 0
      lax.fori_loop(0, CHUNKS // NBUF, deg_iter, 0)
      for s in range(1, NBUF):
        pltpu.make_async_copy(val_b[0], tab_sp.at[dst_b[s]],
                              sc_sems[s]).wait()

    fill(val_b[0], CH, ones16)
    fill(val_b[1], 2 * SUB, zeros16)
    zero_my_slice(val_b[1].at[pl.ds(SUB, SUB)])
    plsc.subcore_barrier()

    ones_pass(dst_hbm)
    plsc.subcore_barrier()
    pltpu.sync_copy(tab_sp.at[pl.ds(nbase, SLICE)], p_repl.at[pl.ds(0, SLICE)])

    def prep_i(i, _):
      s = pl.ds(i * LANES, LANES)
      p_repl[s] = _rsqrt(jnp.maximum(p_repl[s], 1.0))
      return 0
    lax.fori_loop(0, SLICE // LANES, prep_i, 0)
    pltpu.sync_copy(p_repl.at[pl.ds(0, SLICE)], ni_out.at[pl.ds(nbase, SLICE)])
    fill(val_b[1], 2 * SUB, zeros16)
    zero_my_slice(val_b[1].at[pl.ds(SUB, SUB)])
    plsc.subcore_barrier()

    ones_pass(src_hbm)
    plsc.subcore_barrier()
    pltpu.sync_copy(tab_sp.at[pl.ds(nbase, SLICE)], p_repl.at[pl.ds(0, SLICE)])
    pltpu.sync_copy(h_hbm.at[pl.ds(nbase, SLICE)],
                    p_repl.at[pl.ds(SLICE, SLICE)])

    def prep_o(i, _):
      s = pl.ds(i * LANES, LANES)
      ss = pl.ds(SLICE + (i * LANES), LANES)
      no = _rsqrt(jnp.maximum(p_repl[s], 1.0))
      p_repl[s] = no
      p_repl[ss] = p_repl[ss] * no
      return 0
    lax.fori_loop(0, SLICE // LANES, prep_o, 0)
    pltpu.sync_copy(p_repl.at[pl.ds(0, SLICE)], no_out.at[pl.ds(nbase, SLICE)])
    pltpu.sync_copy(p_repl.at[pl.ds(SLICE, SLICE)],
                    tab_sp.at[pl.ds(nbase, SLICE)])
    plsc.subcore_barrier()

    for layer in range(L):
      pltpu.sync_copy(tab_sp.at[pl.ds(0, N)], p_repl)
      plsc.subcore_barrier()
      fill(val_b[1], SUB, zeros16)
      zero_my_slice(val_b[1].at[pl.ds(0, SUB)])
      plsc.subcore_barrier()

      pltpu.async_copy(src_hbm.at[pl.ds(ebase, CH)], src_b[0], idx_sems[0])
      pltpu.async_copy(dst_hbm.at[pl.ds(ebase, CH)], dst_b[0], idx_sems[0])

      def edge_iter(gi, _):
        for u in range(NBUF):
          g = gi * NBUF + u
          un = (u + 1) % NBUF
          v = u % 2

          @pl.when(g >= 2)
          def _reclaim():
            pltpu.make_async_copy(val_b[v], tab_sp.at[dst_b[(u + 2) % 4]],
                                  v_sems[v]).wait()

          @pl.when(g + 1 < CHUNKS)
          def _prefetch():
            off = ebase + (g + 1) * CH
            pltpu.async_copy(src_hbm.at[pl.ds(off, CH)], src_b[un],
                             idx_sems[un])
            pltpu.async_copy(dst_hbm.at[pl.ds(off, CH)], dst_b[un],
                             idx_sems[un])

          pltpu.make_async_copy(src_hbm.at[pl.ds(0, CH)], src_b[u],
                                idx_sems[u]).wait()
          pltpu.make_async_copy(dst_hbm.at[pl.ds(0, CH)], dst_b[u],
                                idx_sems[u]).wait()

          def vgather(j, _):
            for t in range(GU):
              s = pl.ds((j * GU + t) * LANES, LANES)
              val_b[v][s] = plsc.load_gather(p_repl, [src_b[u][s]])
            return 0
          lax.fori_loop(0, CH // (LANES * GU), vgather, 0)

          pltpu.async_copy(val_b[v], tab_sp.at[dst_b[u]], v_sems[v],
                           add=True)
        return 0
      lax.fori_loop(0, CHUNKS // NBUF, edge_iter, 0)
      for v in range(2):
        pltpu.make_async_copy(val_b[v], tab_sp.at[dst_b[2 + v]],
                              v_sems[v]).wait()
      plsc.subcore_barrier()

      c = zeros16
      for q in range(NSUB):
        off = nbase + q * SUB
        pltpu.sync_copy(tab_sp.at[pl.ds(off, SUB)], val_b[0].at[pl.ds(0, SUB)])
        pltpu.sync_copy(ni_out.at[pl.ds(off, SUB)],
                        val_b[0].at[pl.ds(SUB, SUB)])
        pltpu.sync_copy(no_out.at[pl.ds(off, SUB)],
                        val_b[1].at[pl.ds(0, SUB)])

        def node(i, cc):
          s = pl.ds(i * LANES, LANES)
          ss = pl.ds(SUB + (i * LANES), LANES)
          hn = val_b[0][s] * val_b[0][ss]
          val_b[1][ss] = hn * val_b[1][s]
          return cc + hn * hn
        c = lax.fori_loop(0, VPS, node, c)
        pltpu.sync_copy(val_b[1].at[pl.ds(SUB, SUB)],
                        tab_sp.at[pl.ds(off, SUB)])
      cbuf[...] = c
      pltpu.sync_copy(cbuf, c_sp.at[pl.ds(sid * LANES, LANES)])
      plsc.subcore_barrier()

      @pl.when(sid == 0)
      def _reduce():
        pltpu.sync_copy(c_sp, val_b[0].at[pl.ds(0, TILES * LANES)])
        acc = zeros16
        for r in range(TILES):
          acc = acc + val_b[0][pl.ds(r * LANES, LANES)]
        cbuf[...] = acc
        pltpu.sync_copy(cbuf, c_out.at[pl.ds(layer * LANES, LANES)])


@functools.partial(jax.jit, static_argnums=())
def _sc_call(h1, src1d, dst1d):
  mesh = plsc.VectorSubcoreMesh(core_axis_name="c", subcore_axis_name="s")
  f = pl.kernel(
      _sc_body,
      out_type=(
          jax.ShapeDtypeStruct((L * LANES,), jnp.float32),
          jax.ShapeDtypeStruct((NP,), jnp.float32),
          jax.ShapeDtypeStruct((NP,), jnp.float32),
      ),
      mesh=mesh,
      compiler_params=pltpu.CompilerParams(needs_layout_passes=False),
      scratch_types=[
          pltpu.VMEM_SHARED((NP,), jnp.float32),
          pltpu.VMEM_SHARED((TILES * LANES,), jnp.float32),
          [pltpu.VMEM((CH,), jnp.int32)] * NBUF,
          [pltpu.VMEM((CH,), jnp.int32)] * NBUF,
          [pltpu.VMEM((CH,), jnp.float32)] * 2,
          pltpu.VMEM((N,), jnp.float32),
          pltpu.VMEM((LANES,), jnp.float32),
          [pltpu.SemaphoreType.DMA] * NBUF,
          [pltpu.SemaphoreType.DMA] * NBUF,
          [pltpu.SemaphoreType.DMA] * 2,
      ],
  )
  return f(h1, src1d, dst1d)


def kernel(h, edge_index):
  h1 = jnp.pad(h[:, 0], (0, NP - N))
  out, _, _ = _sc_call(h1, edge_index[0], edge_index[1])
  return jnp.sum(out.reshape(L, LANES), axis=1)
